# baseline (speedup 1.0000x reference)
import functools

import jax
import jax.numpy as jnp
from jax import lax
from jax.experimental import pallas as pl
from jax.experimental.pallas import tpu as pltpu
from jax.experimental.pallas import tpu_sc as plsc

_N = 10000
_E = 160000
_H = 128
_R = 8
_B = 2

_NC = 2
_NS = 16
_NW = _NC * _NS

_CH = 64
_DEPTH = 3
_EPT = _E // _NW
_NCH = (_EPT + _CH - 1) // _CH
_NCH += (-_NCH) % _DEPTH
_EPAD = _NCH * _CH
_NPAD = _N + _NS
_ZR = 624
_ZTAIL = _NPAD - _ZR * _NS
_DR = 624
_DTAIL = _N - _DR * _NS

_BLK = 1000


def _sc_body(*refs):
    (P_hbm, gix_hbm, dst_hbm, z_hbm, out_hbm, dst_v, gidx_v) = refs[:7]
    rows = refs[7:7 + _DEPTH]
    agg = refs[7 + _DEPTH]
    sems = refs[8 + _DEPTH:8 + 2 * _DEPTH]
    cid = lax.axis_index("c")
    sid = lax.axis_index("s")
    w = cid * _NS + sid

    pltpu.sync_copy(gix_hbm.at[w], gidx_v)
    pltpu.sync_copy(dst_hbm.at[w], dst_v)

    pltpu.sync_copy(z_hbm, agg.at[pl.ds(sid * _ZR, _ZR)])

    @pl.when(sid == _NS - 1)
    def _():
        pltpu.sync_copy(z_hbm.at[pl.ds(0, _ZTAIL)],
                        agg.at[pl.ds(_NS * _ZR, _ZTAIL)])

    plsc.subcore_barrier()

    for b in range(_DEPTH):
        pltpu.async_copy(P_hbm.at[gidx_v.at[b]], rows[b], sems[b])

    def ring(g, carry):
        base = g * _DEPTH
        for b in range(_DEPTH):
            i = base + b
            pltpu.make_async_copy(P_hbm.at[gidx_v.at[i]], rows[b], sems[b]).wait()
            pltpu.sync_copy(rows[b], agg.at[dst_v.at[i]], add=True)

            @pl.when(g < _NCH // _DEPTH - 1)
            def _():
                pltpu.async_copy(P_hbm.at[gidx_v.at[i + _DEPTH]], rows[b], sems[b])

        return carry

    lax.fori_loop(0, _NCH // _DEPTH, ring, 0)

    plsc.subcore_barrier()
    pltpu.sync_copy(agg.at[pl.ds(sid * _DR, _DR)],
                    out_hbm.at[cid, pl.ds(sid * _DR, _DR)])

    @pl.when(sid == _NS - 1)
    def _():
        pltpu.sync_copy(agg.at[pl.ds(_NS * _DR, _DTAIL)],
                        out_hbm.at[cid, pl.ds(_NS * _DR, _DTAIL)])


def _sc_scatter(P, gixp, dstp, zrows):
    mesh = plsc.VectorSubcoreMesh(core_axis_name="c", subcore_axis_name="s")
    kern = pl.kernel(
        _sc_body,
        out_type=jax.ShapeDtypeStruct((_NC, _N, _H), jnp.float32),
        mesh=mesh,
        scratch_types=(
            [pltpu.VMEM((_NCH, _CH), jnp.int32),
             pltpu.VMEM((_NCH, _CH), jnp.int32)]
            + [pltpu.VMEM((_CH, _H), jnp.float32) for _ in range(_DEPTH)]
            + [pltpu.VMEM_SHARED((_NPAD, _H), jnp.float32)]
            + [pltpu.SemaphoreType.DMA for _ in range(_DEPTH)]
        ),
    )
    return kern(P, gixp, dstp, zrows)


def _gidx_prep(srcp2d, typp2d):
    def body(s_ref, t_ref, g_ref):
        g_ref[...] = s_ref[...] * _R + t_ref[...]

    return pl.pallas_call(
        body,
        out_shape=jax.ShapeDtypeStruct((_NW, _EPAD), jnp.int32),
    )(srcp2d, typp2d)


def _wprep(comb, V):
    def body(comb_ref, V_ref, W_ref):
        for r in range(_R):
            acc = comb_ref[r, 0] * V_ref[0]
            for b in range(1, _B):
                acc = acc + comb_ref[r, b] * V_ref[b]
            W_ref[:, r * _H:(r + 1) * _H] = acc

    return pl.pallas_call(
        body,
        in_specs=[
            pl.BlockSpec(memory_space=pltpu.SMEM),
            pl.BlockSpec((_B, _H, _H), lambda: (0, 0, 0)),
        ],
        out_specs=pl.BlockSpec((_H, _R * _H), lambda: (0, 0)),
        out_shape=jax.ShapeDtypeStruct((_H, _R * _H), jnp.float32),
    )(comb, V)


def _proj(h_args, Wcat, Wself, b2d, fuse_relu):
    nblk = _N // _BLK

    def body(*refs):
        if fuse_relu:
            a_ref, s0_ref, Wc_ref, Ws_ref, b_ref, P_ref, S_ref = refs
            hb = jnp.maximum(a_ref[0] + a_ref[1] + s0_ref[...], 0.0)
        else:
            h_ref, Wc_ref, Ws_ref, b_ref, P_ref, S_ref = refs
            hb = h_ref[...]
        P_ref[...] = jnp.dot(hb, Wc_ref[...], preferred_element_type=jnp.float32)
        S_ref[...] = jnp.dot(hb, Ws_ref[...], preferred_element_type=jnp.float32) + b_ref[...]

    if fuse_relu:
        in_specs = [
            pl.BlockSpec((_NC, _BLK, _H), lambda i: (0, i, 0)),
            pl.BlockSpec((_BLK, _H), lambda i: (i, 0)),
        ]
    else:
        in_specs = [pl.BlockSpec((_BLK, _H), lambda i: (i, 0))]
    in_specs += [
        pl.BlockSpec((_H, _R * _H), lambda i: (0, 0)),
        pl.BlockSpec((_H, _H), lambda i: (0, 0)),
        pl.BlockSpec((1, _H), lambda i: (0, 0)),
    ]
    return pl.pallas_call(
        body,
        grid=(nblk,),
        in_specs=in_specs,
        out_specs=[
            pl.BlockSpec((_BLK, _R * _H), lambda i: (i, 0)),
            pl.BlockSpec((_BLK, _H), lambda i: (i, 0)),
        ],
        out_shape=[
            jax.ShapeDtypeStruct((_N, _R * _H), jnp.float32),
            jax.ShapeDtypeStruct((_N, _H), jnp.float32),
        ],
    )(*h_args, Wcat, Wself, b2d)


def _final(aggp, S1, fcw2d, fcb2d):
    nblk = _N // _BLK

    def body(a_ref, s_ref, f_ref, fcb_ref, o_ref):
        i = pl.program_id(0)
        h2 = a_ref[0] + a_ref[1] + s_ref[...]
        part = jnp.sum(h2 * f_ref[...])

        @pl.when(i == 0)
        def _():
            o_ref[...] = jnp.zeros_like(o_ref)

        o_ref[...] += part

        @pl.when(i == nblk - 1)
        def _():
            x = o_ref[...] + fcb_ref[0]
            o_ref[...] = 1.0 / (1.0 + jnp.exp(-x))

    return pl.pallas_call(
        body,
        grid=(nblk,),
        in_specs=[
            pl.BlockSpec((_NC, _BLK, _H), lambda i: (0, i, 0)),
            pl.BlockSpec((_BLK, _H), lambda i: (i, 0)),
            pl.BlockSpec((_BLK, _H), lambda i: (i, 0)),
            pl.BlockSpec(memory_space=pltpu.SMEM),
        ],
        out_specs=pl.BlockSpec((1, 1), lambda i: (0, 0)),
        out_shape=jax.ShapeDtypeStruct((1, 1), jnp.float32),
    )(aggp, S1, fcw2d, fcb2d)


def kernel(features, edge_index, edge_type, V0, comb0, Wself0, b0,
           V1, comb1, Wself1, b1, fcW, fcb):
    src = edge_index[0]
    dst = edge_index[1]

    pad = _EPAD - _EPT
    srcp = jnp.pad(src.reshape(_NW, _EPT), ((0, 0), (0, pad)))
    typp = jnp.pad(edge_type.reshape(_NW, _EPT), ((0, 0), (0, pad)))
    dstp = jnp.pad(dst.reshape(_NW, _EPT), ((0, 0), (0, pad)),
                   constant_values=_N).reshape(_NW, _NCH, _CH)
    gixp = _gidx_prep(srcp, typp).reshape(_NW, _NCH, _CH)
    zrows = jnp.zeros((_ZR, _H), jnp.float32)

    Wcat0 = _wprep(comb0, V0)
    P0, S0 = _proj((features,), Wcat0, Wself0, b0.reshape(1, _H), fuse_relu=False)
    agg0 = _sc_scatter(P0.reshape(_N * _R, _H), gixp, dstp, zrows)

    Wcat1 = _wprep(comb1, V1)
    P1, S1 = _proj((agg0, S0), Wcat1, Wself1, b1.reshape(1, _H), fuse_relu=True)
    agg1 = _sc_scatter(P1.reshape(_N * _R, _H), gixp, dstp, zrows)

    return _final(agg1, S1, fcW.reshape(_N, _H), fcb)


# speedup vs baseline: 1.2545x; 1.2545x over previous
import functools

import jax
import jax.numpy as jnp
from jax import lax
from jax.experimental import pallas as pl
from jax.experimental.pallas import tpu as pltpu
from jax.experimental.pallas import tpu_sc as plsc

_N = 10000
_E = 160000
_H = 128
_R = 8
_B = 2

_NC = 2
_NS = 16
_NW = _NC * _NS

_CH = 128
_DEPTH = 2
_EPT = _E // _NW
_NCH = (_EPT + _CH - 1) // _CH
_NCH += (-_NCH) % _DEPTH
_EPAD = _NCH * _CH
_NPAD = _N + _NS
_ZR = 624
_ZTAIL = _NPAD - _ZR * _NS
_DR = 624
_DTAIL = _N - _DR * _NS

_BLK = 1000


def _sc_body(*refs):
    (P_hbm, gix_hbm, dst_hbm, z_hbm, out_hbm, dst_v, gidx_v) = refs[:7]
    rows = refs[7:7 + _DEPTH]
    agg = refs[7 + _DEPTH]
    sems = refs[8 + _DEPTH:8 + 2 * _DEPTH]
    cid = lax.axis_index("c")
    sid = lax.axis_index("s")
    w = cid * _NS + sid

    pltpu.sync_copy(gix_hbm.at[w], gidx_v)
    pltpu.sync_copy(dst_hbm.at[w], dst_v)

    pltpu.sync_copy(z_hbm, agg.at[pl.ds(sid * _ZR, _ZR)])

    @pl.when(sid == _NS - 1)
    def _():
        pltpu.sync_copy(z_hbm.at[pl.ds(0, _ZTAIL)],
                        agg.at[pl.ds(_NS * _ZR, _ZTAIL)])

    plsc.subcore_barrier()

    for b in range(_DEPTH):
        pltpu.async_copy(P_hbm.at[gidx_v.at[b]], rows[b], sems[b])

    def ring(g, carry):
        base = g * _DEPTH
        for b in range(_DEPTH):
            i = base + b
            pltpu.make_async_copy(P_hbm.at[gidx_v.at[i]], rows[b], sems[b]).wait()
            pltpu.sync_copy(rows[b], agg.at[dst_v.at[i]], add=True)

            @pl.when(g < _NCH // _DEPTH - 1)
            def _():
                pltpu.async_copy(P_hbm.at[gidx_v.at[i + _DEPTH]], rows[b], sems[b])

        return carry

    lax.fori_loop(0, _NCH // _DEPTH, ring, 0)

    plsc.subcore_barrier()
    pltpu.sync_copy(agg.at[pl.ds(sid * _DR, _DR)],
                    out_hbm.at[cid, pl.ds(sid * _DR, _DR)])

    @pl.when(sid == _NS - 1)
    def _():
        pltpu.sync_copy(agg.at[pl.ds(_NS * _DR, _DTAIL)],
                        out_hbm.at[cid, pl.ds(_NS * _DR, _DTAIL)])


def _sc_scatter(P, gixp, dstp, zrows):
    mesh = plsc.VectorSubcoreMesh(core_axis_name="c", subcore_axis_name="s")
    kern = pl.kernel(
        _sc_body,
        out_type=jax.ShapeDtypeStruct((_NC, _N, _H), jnp.float32),
        mesh=mesh,
        scratch_types=(
            [pltpu.VMEM((_NCH, _CH), jnp.int32),
             pltpu.VMEM((_NCH, _CH), jnp.int32)]
            + [pltpu.VMEM((_CH, _H), jnp.float32) for _ in range(_DEPTH)]
            + [pltpu.VMEM_SHARED((_NPAD, _H), jnp.float32)]
            + [pltpu.SemaphoreType.DMA for _ in range(_DEPTH)]
        ),
    )
    return kern(P, gixp, dstp, zrows)


def _gidx_prep(srcp2d, typp2d):
    def body(s_ref, t_ref, g_ref):
        g_ref[...] = s_ref[...] * _R + t_ref[...]

    return pl.pallas_call(
        body,
        out_shape=jax.ShapeDtypeStruct((_NW, _EPAD), jnp.int32),
    )(srcp2d, typp2d)


def _proj(h_args, comb, V, Wself, b2d, fuse_relu):
    nblk = _N // _BLK

    def body(*refs):
        if fuse_relu:
            a_ref, s0_ref, comb_ref, V_ref, Ws_ref, b_ref, P_ref, S_ref = refs
            hb = jnp.maximum(a_ref[0] + a_ref[1] + s0_ref[...], 0.0)
        else:
            h_ref, comb_ref, V_ref, Ws_ref, b_ref, P_ref, S_ref = refs
            hb = h_ref[...]
        for r in range(_R):
            acc = comb_ref[r, 0] * V_ref[0]
            for b in range(1, _B):
                acc = acc + comb_ref[r, b] * V_ref[b]
            P_ref[:, r * _H:(r + 1) * _H] = jnp.dot(
                hb, acc, preferred_element_type=jnp.float32)
        S_ref[...] = jnp.dot(hb, Ws_ref[...],
                             preferred_element_type=jnp.float32) + b_ref[...]

    if fuse_relu:
        in_specs = [
            pl.BlockSpec((_NC, _BLK, _H), lambda i: (0, i, 0)),
            pl.BlockSpec((_BLK, _H), lambda i: (i, 0)),
        ]
    else:
        in_specs = [pl.BlockSpec((_BLK, _H), lambda i: (i, 0))]
    in_specs += [
        pl.BlockSpec(memory_space=pltpu.SMEM),
        pl.BlockSpec((_B, _H, _H), lambda i: (0, 0, 0)),
        pl.BlockSpec((_H, _H), lambda i: (0, 0)),
        pl.BlockSpec((1, _H), lambda i: (0, 0)),
    ]
    return pl.pallas_call(
        body,
        grid=(nblk,),
        in_specs=in_specs,
        out_specs=[
            pl.BlockSpec((_BLK, _R * _H), lambda i: (i, 0)),
            pl.BlockSpec((_BLK, _H), lambda i: (i, 0)),
        ],
        out_shape=[
            jax.ShapeDtypeStruct((_N, _R * _H), jnp.float32),
            jax.ShapeDtypeStruct((_N, _H), jnp.float32),
        ],
    )(*h_args, comb, V, Wself, b2d)


def _final(aggp, S1, fcw2d, fcb):
    nblk = _N // _BLK

    def body(a_ref, s_ref, f_ref, fcb_ref, o_ref):
        i = pl.program_id(0)
        h2 = a_ref[0] + a_ref[1] + s_ref[...]
        part = jnp.sum(h2 * f_ref[...])

        @pl.when(i == 0)
        def _():
            o_ref[...] = jnp.zeros_like(o_ref)

        o_ref[...] += part

        @pl.when(i == nblk - 1)
        def _():
            x = o_ref[...] + fcb_ref[0]
            o_ref[...] = 1.0 / (1.0 + jnp.exp(-x))

    return pl.pallas_call(
        body,
        grid=(nblk,),
        in_specs=[
            pl.BlockSpec((_NC, _BLK, _H), lambda i: (0, i, 0)),
            pl.BlockSpec((_BLK, _H), lambda i: (i, 0)),
            pl.BlockSpec((_BLK, _H), lambda i: (i, 0)),
            pl.BlockSpec(memory_space=pltpu.SMEM),
        ],
        out_specs=pl.BlockSpec((1, 1), lambda i: (0, 0)),
        out_shape=jax.ShapeDtypeStruct((1, 1), jnp.float32),
    )(aggp, S1, fcw2d, fcb)


def kernel(features, edge_index, edge_type, V0, comb0, Wself0, b0,
           V1, comb1, Wself1, b1, fcW, fcb):
    src = edge_index[0]
    dst = edge_index[1]

    pad = _EPAD - _EPT
    srcp = jnp.pad(src.reshape(_NW, _EPT), ((0, 0), (0, pad)))
    typp = jnp.pad(edge_type.reshape(_NW, _EPT), ((0, 0), (0, pad)))
    dstp = jnp.pad(dst.reshape(_NW, _EPT), ((0, 0), (0, pad)),
                   constant_values=_N).reshape(_NW, _NCH, _CH)
    gixp = _gidx_prep(srcp, typp).reshape(_NW, _NCH, _CH)
    zrows = jnp.zeros((_ZR, _H), jnp.float32)

    P0, S0 = _proj((features,), comb0, V0, Wself0, b0.reshape(1, _H),
                   fuse_relu=False)
    agg0 = _sc_scatter(P0.reshape(_N * _R, _H), gixp, dstp, zrows)

    P1, S1 = _proj((agg0, S0), comb1, V1, Wself1, b1.reshape(1, _H),
                   fuse_relu=True)
    agg1 = _sc_scatter(P1.reshape(_N * _R, _H), gixp, dstp, zrows)

    return _final(agg1, S1, fcW.reshape(_N, _H), fcb)


# speedup vs baseline: 1.4600x; 1.1638x over previous
import functools

import jax
import jax.numpy as jnp
from jax import lax
from jax.experimental import pallas as pl
from jax.experimental.pallas import tpu as pltpu
from jax.experimental.pallas import tpu_sc as plsc

_N = 10000
_E = 160000
_H = 128
_R = 8
_B = 2

_NC = 2
_NS = 16
_NW = _NC * _NS

_CH = 128
_DEPTH = 2
_EPT = _E // _NW
_NCH = (_EPT + _CH - 1) // _CH
_NCH += (-_NCH) % _DEPTH
_EPAD = _NCH * _CH
_NPAD = _N + _NS
_ZR = 624
_ZTAIL = _NPAD - _ZR * _NS
_DR = 624
_DTAIL = _N - _DR * _NS

_BLK = 1000


def _sc_body(*refs):
    (P_hbm, gix_hbm, dst_hbm, z_hbm, out_hbm, dst_v, gidx_v) = refs[:7]
    rows = refs[7:7 + _DEPTH]
    agg = refs[7 + _DEPTH]
    sems = refs[8 + _DEPTH:8 + 2 * _DEPTH]
    cid = lax.axis_index("c")
    sid = lax.axis_index("s")
    w = cid * _NS + sid

    pltpu.sync_copy(gix_hbm.at[w], gidx_v)
    pltpu.sync_copy(dst_hbm.at[w], dst_v)

    pltpu.sync_copy(z_hbm, agg.at[pl.ds(sid * _ZR, _ZR)])

    @pl.when(sid == _NS - 1)
    def _():
        pltpu.sync_copy(z_hbm.at[pl.ds(0, _ZTAIL)],
                        agg.at[pl.ds(_NS * _ZR, _ZTAIL)])

    plsc.subcore_barrier()

    for b in range(_DEPTH):
        pltpu.async_copy(P_hbm.at[gidx_v.at[b]], rows[b], sems[b])

    def ring(g, carry):
        base = g * _DEPTH
        for b in range(_DEPTH):
            i = base + b
            pltpu.make_async_copy(P_hbm.at[gidx_v.at[i]], rows[b], sems[b]).wait()
            pltpu.sync_copy(rows[b], agg.at[dst_v.at[i]], add=True)

            @pl.when(g < _NCH // _DEPTH - 1)
            def _():
                pltpu.async_copy(P_hbm.at[gidx_v.at[i + _DEPTH]], rows[b], sems[b])

        return carry

    lax.fori_loop(0, _NCH // _DEPTH, ring, 0)

    plsc.subcore_barrier()
    pltpu.sync_copy(agg.at[pl.ds(sid * _DR, _DR)],
                    out_hbm.at[cid, pl.ds(sid * _DR, _DR)])

    @pl.when(sid == _NS - 1)
    def _():
        pltpu.sync_copy(agg.at[pl.ds(_NS * _DR, _DTAIL)],
                        out_hbm.at[cid, pl.ds(_NS * _DR, _DTAIL)])


def _sc_scatter(P, gixp, dstp, zrows):
    mesh = plsc.VectorSubcoreMesh(core_axis_name="c", subcore_axis_name="s")
    kern = pl.kernel(
        _sc_body,
        out_type=jax.ShapeDtypeStruct((_NC, _N, _H), jnp.float32),
        mesh=mesh,
        scratch_types=(
            [pltpu.VMEM((_NCH, _CH), jnp.int32),
             pltpu.VMEM((_NCH, _CH), jnp.int32)]
            + [pltpu.VMEM((_CH, _H), jnp.float32) for _ in range(_DEPTH)]
            + [pltpu.VMEM_SHARED((_NPAD, _H), jnp.float32)]
            + [pltpu.SemaphoreType.DMA for _ in range(_DEPTH)]
        ),
    )
    return kern(P, gixp, dstp, zrows)


def _gidx_prep(srcp2d, typp2d):
    def body(s_ref, t_ref, g_ref):
        g_ref[...] = t_ref[...] * _N + s_ref[...]

    return pl.pallas_call(
        body,
        out_shape=jax.ShapeDtypeStruct((_NW, _EPAD), jnp.int32),
    )(srcp2d, typp2d)


def _proj(h_args, comb, V, Wself, b2d, fuse_relu):
    nblk = _N // _BLK

    def body(*refs):
        if fuse_relu:
            a_ref, s0_ref, comb_ref, V_ref, Ws_ref, b_ref, P_ref, S_ref = refs
            hb = jnp.maximum(a_ref[0] + a_ref[1] + s0_ref[...], 0.0)
        else:
            h_ref, comb_ref, V_ref, Ws_ref, b_ref, P_ref, S_ref = refs
            hb = h_ref[...]
        for r in range(_R):
            acc = comb_ref[r, 0] * V_ref[0]
            for b in range(1, _B):
                acc = acc + comb_ref[r, b] * V_ref[b]
            P_ref[r] = jnp.dot(hb, acc, preferred_element_type=jnp.float32)
        S_ref[...] = jnp.dot(hb, Ws_ref[...],
                             preferred_element_type=jnp.float32) + b_ref[...]

    if fuse_relu:
        in_specs = [
            pl.BlockSpec((_NC, _BLK, _H), lambda i: (0, i, 0)),
            pl.BlockSpec((_BLK, _H), lambda i: (i, 0)),
        ]
    else:
        in_specs = [pl.BlockSpec((_BLK, _H), lambda i: (i, 0))]
    in_specs += [
        pl.BlockSpec(memory_space=pltpu.SMEM),
        pl.BlockSpec((_B, _H, _H), lambda i: (0, 0, 0)),
        pl.BlockSpec((_H, _H), lambda i: (0, 0)),
        pl.BlockSpec((1, _H), lambda i: (0, 0)),
    ]
    return pl.pallas_call(
        body,
        grid=(nblk,),
        in_specs=in_specs,
        out_specs=[
            pl.BlockSpec((_R, _BLK, _H), lambda i: (0, i, 0)),
            pl.BlockSpec((_BLK, _H), lambda i: (i, 0)),
        ],
        out_shape=[
            jax.ShapeDtypeStruct((_R, _N, _H), jnp.float32),
            jax.ShapeDtypeStruct((_N, _H), jnp.float32),
        ],
    )(*h_args, comb, V, Wself, b2d)


def _final(aggp, S1, fcw2d, fcb):
    nblk = _N // _BLK

    def body(a_ref, s_ref, f_ref, fcb_ref, o_ref):
        i = pl.program_id(0)
        h2 = a_ref[0] + a_ref[1] + s_ref[...]
        part = jnp.sum(h2 * f_ref[...])

        @pl.when(i == 0)
        def _():
            o_ref[...] = jnp.zeros_like(o_ref)

        o_ref[...] += part

        @pl.when(i == nblk - 1)
        def _():
            x = o_ref[...] + fcb_ref[0]
            o_ref[...] = 1.0 / (1.0 + jnp.exp(-x))

    return pl.pallas_call(
        body,
        grid=(nblk,),
        in_specs=[
            pl.BlockSpec((_NC, _BLK, _H), lambda i: (0, i, 0)),
            pl.BlockSpec((_BLK, _H), lambda i: (i, 0)),
            pl.BlockSpec((_BLK, _H), lambda i: (i, 0)),
            pl.BlockSpec(memory_space=pltpu.SMEM),
        ],
        out_specs=pl.BlockSpec((1, 1), lambda i: (0, 0)),
        out_shape=jax.ShapeDtypeStruct((1, 1), jnp.float32),
    )(aggp, S1, fcw2d, fcb)


def kernel(features, edge_index, edge_type, V0, comb0, Wself0, b0,
           V1, comb1, Wself1, b1, fcW, fcb):
    src = edge_index[0]
    dst = edge_index[1]

    pad = _EPAD - _EPT
    srcp = jnp.pad(src.reshape(_NW, _EPT), ((0, 0), (0, pad)))
    typp = jnp.pad(edge_type.reshape(_NW, _EPT), ((0, 0), (0, pad)))
    dstp = jnp.pad(dst.reshape(_NW, _EPT), ((0, 0), (0, pad)),
                   constant_values=_N).reshape(_NW, _NCH, _CH)
    gixp = _gidx_prep(srcp, typp).reshape(_NW, _NCH, _CH)
    zrows = jnp.zeros((_ZR, _H), jnp.float32)

    P0, S0 = _proj((features,), comb0, V0, Wself0, b0.reshape(1, _H),
                   fuse_relu=False)
    agg0 = _sc_scatter(P0.reshape(_R * _N, _H), gixp, dstp, zrows)

    P1, S1 = _proj((agg0, S0), comb1, V1, Wself1, b1.reshape(1, _H),
                   fuse_relu=True)
    agg1 = _sc_scatter(P1.reshape(_R * _N, _H), gixp, dstp, zrows)

    return _final(agg1, S1, fcW.reshape(_N, _H), fcb)


# speedup vs baseline: 1.4940x; 1.0233x over previous
import functools

import jax
import jax.numpy as jnp
from jax import lax
from jax.experimental import pallas as pl
from jax.experimental.pallas import tpu as pltpu
from jax.experimental.pallas import tpu_sc as plsc

_N = 10000
_E = 160000
_H = 128
_R = 8
_B = 2

_NC = 2
_NS = 16
_NW = _NC * _NS

_CH = 128
_DEPTH = 2
_EPT = _E // _NW
_NCH = (_EPT + _CH - 1) // _CH
_NCH += (-_NCH) % _DEPTH
_EPAD = _NCH * _CH
_NPAD = _N + _NS
_ZR = 624
_ZTAIL = _NPAD - _ZR * _NS
_DR = 624
_DTAIL = _N - _DR * _NS

_BLK = 2000


def _sc_body(*refs):
    (P_hbm, gix_hbm, dst_hbm, z_hbm, out_hbm, dst_v, gidx_v) = refs[:7]
    rows = refs[7:7 + _DEPTH]
    agg = refs[7 + _DEPTH]
    sems = refs[8 + _DEPTH:8 + 2 * _DEPTH]
    cid = lax.axis_index("c")
    sid = lax.axis_index("s")
    w = cid * _NS + sid

    pltpu.sync_copy(gix_hbm.at[w], gidx_v)

    for b in range(_DEPTH):
        pltpu.async_copy(P_hbm.at[gidx_v.at[b]], rows[b], sems[b])

    pltpu.sync_copy(dst_hbm.at[w], dst_v)

    pltpu.sync_copy(z_hbm, agg.at[pl.ds(sid * _ZR, _ZR)])

    @pl.when(sid == _NS - 1)
    def _():
        pltpu.sync_copy(z_hbm.at[pl.ds(0, _ZTAIL)],
                        agg.at[pl.ds(_NS * _ZR, _ZTAIL)])

    plsc.subcore_barrier()

    def ring(g, carry):
        base = g * _DEPTH
        for b in range(_DEPTH):
            i = base + b
            pltpu.make_async_copy(P_hbm.at[gidx_v.at[i]], rows[b], sems[b]).wait()
            pltpu.sync_copy(rows[b], agg.at[dst_v.at[i]], add=True)

            @pl.when(g < _NCH // _DEPTH - 1)
            def _():
                pltpu.async_copy(P_hbm.at[gidx_v.at[i + _DEPTH]], rows[b], sems[b])

        return carry

    lax.fori_loop(0, _NCH // _DEPTH, ring, 0)

    plsc.subcore_barrier()
    pltpu.sync_copy(agg.at[pl.ds(sid * _DR, _DR)],
                    out_hbm.at[cid, pl.ds(sid * _DR, _DR)])

    @pl.when(sid == _NS - 1)
    def _():
        pltpu.sync_copy(agg.at[pl.ds(_NS * _DR, _DTAIL)],
                        out_hbm.at[cid, pl.ds(_NS * _DR, _DTAIL)])


def _sc_scatter(P, gixp, dstp, zrows):
    mesh = plsc.VectorSubcoreMesh(core_axis_name="c", subcore_axis_name="s")
    kern = pl.kernel(
        _sc_body,
        out_type=jax.ShapeDtypeStruct((_NC, _N, _H), jnp.float32),
        mesh=mesh,
        scratch_types=(
            [pltpu.VMEM((_NCH, _CH), jnp.int32),
             pltpu.VMEM((_NCH, _CH), jnp.int32)]
            + [pltpu.VMEM((_CH, _H), jnp.float32) for _ in range(_DEPTH)]
            + [pltpu.VMEM_SHARED((_NPAD, _H), jnp.float32)]
            + [pltpu.SemaphoreType.DMA for _ in range(_DEPTH)]
        ),
    )
    return kern(P, gixp, dstp, zrows)


def _gidx_prep(srcp2d, typp2d):
    def body(s_ref, t_ref, g_ref):
        g_ref[...] = t_ref[...] * _N + s_ref[...]

    return pl.pallas_call(
        body,
        out_shape=jax.ShapeDtypeStruct((_NW, _EPAD), jnp.int32),
    )(srcp2d, typp2d)


def _proj(h_args, comb, V, Wself, b2d, fuse_relu, fcw2d=None):
    nblk = _N // _BLK

    def body(*refs):
        if fuse_relu:
            a_ref, s0_ref, comb_ref, V_ref, Ws_ref, b_ref, f_ref, P_ref, S_ref = refs
            hb = jnp.maximum(a_ref[0] + a_ref[1] + s0_ref[...], 0.0)
        else:
            h_ref, comb_ref, V_ref, Ws_ref, b_ref, P_ref, S_ref = refs
            hb = h_ref[...]
        for r in range(_R):
            acc = comb_ref[r, 0] * V_ref[0]
            for b in range(1, _B):
                acc = acc + comb_ref[r, b] * V_ref[b]
            P_ref[r] = jnp.dot(hb, acc, preferred_element_type=jnp.float32)
        S = jnp.dot(hb, Ws_ref[...],
                    preferred_element_type=jnp.float32) + b_ref[...]
        if fcw2d is None:
            S_ref[...] = S
        else:
            i = pl.program_id(0)
            part = jnp.sum(S * f_ref[...])

            @pl.when(i == 0)
            def _():
                S_ref[...] = jnp.zeros_like(S_ref)

            S_ref[...] += part

    if fuse_relu:
        in_specs = [
            pl.BlockSpec((_NC, _BLK, _H), lambda i: (0, i, 0)),
            pl.BlockSpec((_BLK, _H), lambda i: (i, 0)),
        ]
    else:
        in_specs = [pl.BlockSpec((_BLK, _H), lambda i: (i, 0))]
    in_specs += [
        pl.BlockSpec(memory_space=pltpu.SMEM),
        pl.BlockSpec((_B, _H, _H), lambda i: (0, 0, 0)),
        pl.BlockSpec((_H, _H), lambda i: (0, 0)),
        pl.BlockSpec((1, _H), lambda i: (0, 0)),
    ]
    args = list(h_args) + [comb, V, Wself, b2d]
    if fcw2d is None:
        s_spec = pl.BlockSpec((_BLK, _H), lambda i: (i, 0))
        s_shape = jax.ShapeDtypeStruct((_N, _H), jnp.float32)
    else:
        in_specs.append(pl.BlockSpec((_BLK, _H), lambda i: (i, 0)))
        args.append(fcw2d)
        s_spec = pl.BlockSpec((1, 1), lambda i: (0, 0))
        s_shape = jax.ShapeDtypeStruct((1, 1), jnp.float32)
    return pl.pallas_call(
        body,
        grid=(nblk,),
        in_specs=in_specs,
        out_specs=[
            pl.BlockSpec((_R, _BLK, _H), lambda i: (0, i, 0)),
            s_spec,
        ],
        out_shape=[
            jax.ShapeDtypeStruct((_R, _N, _H), jnp.float32),
            s_shape,
        ],
    )(*args)


def _final(aggp, t1, fcw2d, fcb):
    nblk = _N // _BLK

    def body(a_ref, t_ref, f_ref, fcb_ref, o_ref):
        i = pl.program_id(0)
        part = jnp.sum((a_ref[0] + a_ref[1]) * f_ref[...])

        @pl.when(i == 0)
        def _():
            o_ref[...] = jnp.zeros_like(o_ref)

        o_ref[...] += part

        @pl.when(i == nblk - 1)
        def _():
            x = o_ref[...] + t_ref[...] + fcb_ref[0]
            o_ref[...] = 1.0 / (1.0 + jnp.exp(-x))

    return pl.pallas_call(
        body,
        grid=(nblk,),
        in_specs=[
            pl.BlockSpec((_NC, _BLK, _H), lambda i: (0, i, 0)),
            pl.BlockSpec((1, 1), lambda i: (0, 0)),
            pl.BlockSpec((_BLK, _H), lambda i: (i, 0)),
            pl.BlockSpec(memory_space=pltpu.SMEM),
        ],
        out_specs=pl.BlockSpec((1, 1), lambda i: (0, 0)),
        out_shape=jax.ShapeDtypeStruct((1, 1), jnp.float32),
    )(aggp, t1, fcw2d, fcb)


def kernel(features, edge_index, edge_type, V0, comb0, Wself0, b0,
           V1, comb1, Wself1, b1, fcW, fcb):
    src = edge_index[0]
    dst = edge_index[1]

    pad = _EPAD - _EPT
    srcp = jnp.pad(src.reshape(_NW, _EPT), ((0, 0), (0, pad)))
    typp = jnp.pad(edge_type.reshape(_NW, _EPT), ((0, 0), (0, pad)))
    dstp = jnp.pad(dst.reshape(_NW, _EPT), ((0, 0), (0, pad)),
                   constant_values=_N).reshape(_NW, _NCH, _CH)
    gixp = _gidx_prep(srcp, typp).reshape(_NW, _NCH, _CH)
    zrows = jnp.zeros((_ZR, _H), jnp.float32)

    P0, S0 = _proj((features,), comb0, V0, Wself0, b0.reshape(1, _H),
                   fuse_relu=False)
    agg0 = _sc_scatter(P0.reshape(_R * _N, _H), gixp, dstp, zrows)

    fcw2d = fcW.reshape(_N, _H)
    P1, t1 = _proj((agg0, S0), comb1, V1, Wself1, b1.reshape(1, _H),
                   fuse_relu=True, fcw2d=fcw2d)
    agg1 = _sc_scatter(P1.reshape(_R * _N, _H), gixp, dstp, zrows)

    return _final(agg1, t1, fcw2d, fcb)


# speedup vs baseline: 1.4943x; 1.0002x over previous
import jax
import jax.numpy as jnp
from jax import lax
from jax.experimental import pallas as pl
from jax.experimental.pallas import tpu as pltpu
from jax.experimental.pallas import tpu_sc as plsc

_N = 10000
_E = 160000
_H = 128
_R = 8
_B = 2

_NC = 2
_NS = 16
_NW = _NC * _NS

_CH = 128
_DEPTH = 2
_EPT = _E // _NW
_NCH = (_EPT + _CH - 1) // _CH
_NCH += (-_NCH) % _DEPTH
_EPAD = _NCH * _CH
_NPAD = _N + _NS
_ZR = 624
_ZTAIL = _NPAD - _ZR * _NS
_DR = 624
_DTAIL = _N - _DR * _NS

_BLK = 2000


def _sc_body(*refs):
    (P_hbm, gix_hbm, dst_hbm, z_hbm, out_hbm, dst_v, gidx_v) = refs[:7]
    rows = refs[7:7 + _DEPTH]
    agg = refs[7 + _DEPTH]
    sems = refs[8 + _DEPTH:8 + 2 * _DEPTH]
    cid = lax.axis_index("c")
    sid = lax.axis_index("s")
    w = cid * _NS + sid

    pltpu.sync_copy(gix_hbm.at[w], gidx_v)

    for b in range(_DEPTH):
        pltpu.async_copy(P_hbm.at[gidx_v.at[b]], rows[b], sems[b])

    pltpu.sync_copy(dst_hbm.at[w], dst_v)

    pltpu.sync_copy(z_hbm, agg.at[pl.ds(sid * _ZR, _ZR)])

    @pl.when(sid == _NS - 1)
    def _():
        pltpu.sync_copy(z_hbm.at[pl.ds(0, _ZTAIL)],
                        agg.at[pl.ds(_NS * _ZR, _ZTAIL)])

    plsc.subcore_barrier()

    def ring(g, carry):
        base = g * _DEPTH
        for b in range(_DEPTH):
            i = base + b
            pltpu.make_async_copy(P_hbm.at[gidx_v.at[i]], rows[b], sems[b]).wait()
            pltpu.sync_copy(rows[b], agg.at[dst_v.at[i]], add=True)

            @pl.when(g < _NCH // _DEPTH - 1)
            def _():
                pltpu.async_copy(P_hbm.at[gidx_v.at[i + _DEPTH]], rows[b], sems[b])

        return carry

    lax.fori_loop(0, _NCH // _DEPTH, ring, 0)

    plsc.subcore_barrier()
    pltpu.sync_copy(agg.at[pl.ds(sid * _DR, _DR)],
                    out_hbm.at[cid, pl.ds(sid * _DR, _DR)])

    @pl.when(sid == _NS - 1)
    def _():
        pltpu.sync_copy(agg.at[pl.ds(_NS * _DR, _DTAIL)],
                        out_hbm.at[cid, pl.ds(_NS * _DR, _DTAIL)])


def _sc_scatter(P, gixp, dstp, zrows):
    mesh = plsc.VectorSubcoreMesh(core_axis_name="c", subcore_axis_name="s")
    kern = pl.kernel(
        _sc_body,
        out_type=jax.ShapeDtypeStruct((_NC, _N, _H), jnp.float32),
        mesh=mesh,
        scratch_types=(
            [pltpu.VMEM((_NCH, _CH), jnp.int32),
             pltpu.VMEM((_NCH, _CH), jnp.int32)]
            + [pltpu.VMEM((_CH, _H), jnp.float32) for _ in range(_DEPTH)]
            + [pltpu.VMEM_SHARED((_NPAD, _H), jnp.float32)]
            + [pltpu.SemaphoreType.DMA for _ in range(_DEPTH)]
        ),
    )
    return kern(P, gixp, dstp, zrows)


def _gidx_prep(srcp2d, typp2d):
    def body(s_ref, t_ref, g_ref):
        g_ref[...] = t_ref[...] * _N + s_ref[...]

    return pl.pallas_call(
        body,
        out_shape=jax.ShapeDtypeStruct((_NW, _EPAD), jnp.int32),
    )(srcp2d, typp2d)


def _proj(h_args, comb, V, Wself, b2d, fuse_relu, fcw2d=None):
    nblk = _N // _BLK

    def body(*refs):
        if fuse_relu:
            a_ref, s0_ref, comb_ref, V_ref, Ws_ref, b_ref, f_ref, P_ref, S_ref = refs
            hb = jnp.maximum(a_ref[0] + a_ref[1] + s0_ref[...], 0.0)
        else:
            h_ref, comb_ref, V_ref, Ws_ref, b_ref, P_ref, S_ref = refs
            hb = h_ref[...]
        for r in range(_R):
            acc = comb_ref[r, 0] * V_ref[0]
            for b in range(1, _B):
                acc = acc + comb_ref[r, b] * V_ref[b]
            P_ref[r] = jnp.dot(hb, acc, preferred_element_type=jnp.float32)
        S = jnp.dot(hb, Ws_ref[...],
                    preferred_element_type=jnp.float32) + b_ref[...]
        if fcw2d is None:
            S_ref[...] = S
        else:
            i = pl.program_id(0)
            part = jnp.sum(S * f_ref[...])

            @pl.when(i == 0)
            def _():
                S_ref[...] = jnp.zeros_like(S_ref)

            S_ref[...] += part

    if fuse_relu:
        in_specs = [
            pl.BlockSpec((_NC, _BLK, _H), lambda i: (0, i, 0)),
            pl.BlockSpec((_BLK, _H), lambda i: (i, 0)),
        ]
    else:
        in_specs = [pl.BlockSpec((_BLK, _H), lambda i: (i, 0))]
    in_specs += [
        pl.BlockSpec(memory_space=pltpu.SMEM),
        pl.BlockSpec((_B, _H, _H), lambda i: (0, 0, 0)),
        pl.BlockSpec((_H, _H), lambda i: (0, 0)),
        pl.BlockSpec((1, _H), lambda i: (0, 0)),
    ]
    args = list(h_args) + [comb, V, Wself, b2d]
    if fcw2d is None:
        s_spec = pl.BlockSpec((_BLK, _H), lambda i: (i, 0))
        s_shape = jax.ShapeDtypeStruct((_N, _H), jnp.float32)
    else:
        in_specs.append(pl.BlockSpec((_BLK, _H), lambda i: (i, 0)))
        args.append(fcw2d)
        s_spec = pl.BlockSpec((1, 1), lambda i: (0, 0))
        s_shape = jax.ShapeDtypeStruct((1, 1), jnp.float32)
    return pl.pallas_call(
        body,
        grid=(nblk,),
        in_specs=in_specs,
        out_specs=[
            pl.BlockSpec((_R, _BLK, _H), lambda i: (0, i, 0)),
            s_spec,
        ],
        out_shape=[
            jax.ShapeDtypeStruct((_R, _N, _H), jnp.float32),
            s_shape,
        ],
    )(*args)


def _final(aggp, t1, fcw2d, fcb):
    nblk = _N // _BLK

    def body(a_ref, t_ref, f_ref, fcb_ref, o_ref):
        i = pl.program_id(0)
        part = jnp.sum((a_ref[0] + a_ref[1]) * f_ref[...])

        @pl.when(i == 0)
        def _():
            o_ref[...] = jnp.zeros_like(o_ref)

        o_ref[...] += part

        @pl.when(i == nblk - 1)
        def _():
            x = o_ref[...] + t_ref[...] + fcb_ref[0]
            o_ref[...] = 1.0 / (1.0 + jnp.exp(-x))

    return pl.pallas_call(
        body,
        grid=(nblk,),
        in_specs=[
            pl.BlockSpec((_NC, _BLK, _H), lambda i: (0, i, 0)),
            pl.BlockSpec((1, 1), lambda i: (0, 0)),
            pl.BlockSpec((_BLK, _H), lambda i: (i, 0)),
            pl.BlockSpec(memory_space=pltpu.SMEM),
        ],
        out_specs=pl.BlockSpec((1, 1), lambda i: (0, 0)),
        out_shape=jax.ShapeDtypeStruct((1, 1), jnp.float32),
    )(aggp, t1, fcw2d, fcb)


def kernel(features, edge_index, edge_type, V0, comb0, Wself0, b0,
           V1, comb1, Wself1, b1, fcW, fcb):
    src = edge_index[0]
    dst = edge_index[1]

    pad = _EPAD - _EPT
    srcp = jnp.pad(src.reshape(_NW, _EPT), ((0, 0), (0, pad)))
    typp = jnp.pad(edge_type.reshape(_NW, _EPT), ((0, 0), (0, pad)))
    dstp = jnp.pad(dst.reshape(_NW, _EPT), ((0, 0), (0, pad)),
                   constant_values=_N).reshape(_NW, _NCH, _CH)
    gixp = _gidx_prep(srcp, typp).reshape(_NW, _NCH, _CH)
    zrows = jnp.zeros((_ZR, _H), jnp.float32)

    P0, S0 = _proj((features,), comb0, V0, Wself0, b0.reshape(1, _H),
                   fuse_relu=False)
    agg0 = _sc_scatter(P0.reshape(_R * _N, _H), gixp, dstp, zrows)

    fcw2d = fcW.reshape(_N, _H)
    P1, t1 = _proj((agg0, S0), comb1, V1, Wself1, b1.reshape(1, _H),
                   fuse_relu=True, fcw2d=fcw2d)
    agg1 = _sc_scatter(P1.reshape(_R * _N, _H), gixp, dstp, zrows)

    return _final(agg1, t1, fcw2d, fcb)
